# baseline (speedup 1.0000x reference)
import functools

import jax
import jax.numpy as jnp
from jax import lax
from jax.experimental import pallas as pl
from jax.experimental.pallas import tpu as pltpu
from jax.experimental.pallas import tpu_sc as plsc

_B = 1024
_M = 16
_C = 128
_NNZ = 512
_NT = 34

_LANES = 16
_NW = 32
_BPW = _B // _NW
_NCHUNK = _NNZ // _LANES


def _sc_tensor_product(x1f, x2f, wf, cg, a1, a2, aw, ao):
    mesh = plsc.VectorSubcoreMesh(core_axis_name="c", subcore_axis_name="s")

    @functools.partial(
        pl.kernel,
        mesh=mesh,
        out_type=jax.ShapeDtypeStruct((_B, _M * _C), jnp.float32),
        compiler_params=pltpu.CompilerParams(needs_layout_passes=False),
        scratch_types=[
            pltpu.VMEM((_NNZ,), jnp.int32),
            pltpu.VMEM((_NNZ,), jnp.int32),
            pltpu.VMEM((_NNZ,), jnp.int32),
            pltpu.VMEM((_NNZ,), jnp.int32),
            pltpu.VMEM((_NNZ,), jnp.float32),
            pltpu.VMEM((_M * _C,), jnp.float32),
            pltpu.VMEM((_M * _C,), jnp.float32),
            pltpu.VMEM((_NT * _C,), jnp.float32),
            pltpu.VMEM((_M * _C,), jnp.float32),
        ],
    )
    def k(x1_hbm, x2_hbm, w_hbm, cg_hbm, a1_hbm, a2_hbm, aw_hbm, ao_hbm,
          out_hbm, a1_v, a2_v, aw_v, ao_v, cg_v, x1_v, x2_v, w_v, out_v):
        wid = lax.axis_index("c") * 16 + lax.axis_index("s")

        pltpu.sync_copy(a1_hbm, a1_v)
        pltpu.sync_copy(a2_hbm, a2_v)
        pltpu.sync_copy(aw_hbm, aw_v)
        pltpu.sync_copy(ao_hbm, ao_v)
        pltpu.sync_copy(cg_hbm, cg_v)

        def batch_body(i, carry):
            b = wid * _BPW + i
            pltpu.sync_copy(x1_hbm.at[b], x1_v)
            pltpu.sync_copy(x2_hbm.at[b], x2_v)
            pltpu.sync_copy(w_hbm.at[b], w_v)

            def zero_body(kk, c2):
                out_v[pl.ds(kk * _LANES, _LANES)] = jnp.zeros(
                    (_LANES,), jnp.float32)
                return c2
            lax.fori_loop(0, _M * _C // _LANES, zero_body, 0)

            def chunk_body(nc, c3):
                base = nc * _LANES
                i1 = a1_v[pl.ds(base, _LANES)]
                i2 = a2_v[pl.ds(base, _LANES)]
                iw = aw_v[pl.ds(base, _LANES)]
                io = ao_v[pl.ds(base, _LANES)]
                cgc = cg_v[pl.ds(base, _LANES)]

                def c_body(cq, c4):
                    c0 = cq * 4
                    for d in range(4):
                        c = c0 + d
                        g1 = plsc.load_gather(x1_v, [i1 + c])
                        g2 = plsc.load_gather(x2_v, [i2 + c])
                        gw = plsc.load_gather(w_v, [iw + c])
                        t = g1 * g2 * gw * cgc
                        plsc.addupdate_scatter(out_v, [io + c], t)
                    return c4
                lax.fori_loop(0, _C // 4, c_body, 0)
                return c3
            lax.fori_loop(0, _NCHUNK, chunk_body, 0)

            pltpu.sync_copy(out_v, out_hbm.at[b])
            return carry
        lax.fori_loop(0, _BPW, batch_body, 0)

    return k(x1f, x2f, wf, cg, a1, a2, aw, ao)


def kernel(x1, x2, weight, CG_vals, l_ind_M1M2, M1, M2, M_ptr_M1M2):
    n_idx = jnp.arange(_NNZ, dtype=jnp.int32)
    seg = jnp.sum(n_idx[None, :] >= M_ptr_M1M2[1:_M, None],
                  axis=0, dtype=jnp.int32)
    a1 = M1 * _C
    a2 = M2 * _C
    aw = l_ind_M1M2 * _C
    ao = seg * _C

    x1f = x1.reshape(_B, _M * _C)
    x2f = x2.reshape(_B, _M * _C)
    wf = weight.reshape(_B, _NT * _C)

    out = _sc_tensor_product(x1f, x2f, wf, CG_vals, a1, a2, aw, ao)
    return out.reshape(_B, _M, _C)


# speedup vs baseline: 1.2504x; 1.2504x over previous
import functools

import jax
import jax.numpy as jnp
from jax import lax
from jax.experimental import pallas as pl
from jax.experimental.pallas import tpu as pltpu
from jax.experimental.pallas import tpu_sc as plsc

_B = 1024
_M = 16
_C = 128
_NNZ = 512
_NT = 34

_LANES = 16
_NW = 32
_BPW = _B // _NW
_NCHUNK = _NNZ // _LANES


def _sc_tensor_product(x1f, x2f, wf, cg, a1, a2, aw, ao):
    mesh = plsc.VectorSubcoreMesh(core_axis_name="c", subcore_axis_name="s")

    @functools.partial(
        pl.kernel,
        mesh=mesh,
        out_type=jax.ShapeDtypeStruct((_B, _M * _C), jnp.float32),
        compiler_params=pltpu.CompilerParams(needs_layout_passes=False),
        scratch_types=[
            pltpu.VMEM((_NNZ,), jnp.int32),
            pltpu.VMEM((_NNZ,), jnp.int32),
            pltpu.VMEM((_NNZ,), jnp.int32),
            pltpu.VMEM((_NNZ,), jnp.int32),
            pltpu.VMEM((_NNZ,), jnp.float32),
            pltpu.VMEM((_M * _C,), jnp.float32),
            pltpu.VMEM((_M * _C,), jnp.float32),
            pltpu.VMEM((_NT * _C,), jnp.float32),
            pltpu.VMEM((_M * _C,), jnp.float32),
        ],
    )
    def k(x1_hbm, x2_hbm, w_hbm, cg_hbm, a1_hbm, a2_hbm, aw_hbm, ao_hbm,
          out_hbm, a1_v, a2_v, aw_v, ao_v, cg_v, x1_v, x2_v, w_v, out_v):
        wid = lax.axis_index("c") * 16 + lax.axis_index("s")

        pltpu.sync_copy(a1_hbm, a1_v)
        pltpu.sync_copy(a2_hbm, a2_v)
        pltpu.sync_copy(aw_hbm, aw_v)
        pltpu.sync_copy(ao_hbm, ao_v)
        pltpu.sync_copy(cg_hbm, cg_v)

        def batch_body(i, carry):
            b = wid * _BPW + i
            pltpu.sync_copy(x1_hbm.at[b], x1_v)
            pltpu.sync_copy(x2_hbm.at[b], x2_v)
            pltpu.sync_copy(w_hbm.at[b], w_v)

            @plsc.parallel_loop(0, _M * _C, _LANES, unroll=4)
            def zero_body(kk):
                out_v[pl.ds(kk, _LANES)] = jnp.zeros((_LANES,), jnp.float32)

            def chunk_body(nc, c3):
                base = nc * _LANES
                i1 = a1_v[pl.ds(base, _LANES)]
                i2 = a2_v[pl.ds(base, _LANES)]
                iw = aw_v[pl.ds(base, _LANES)]
                io = ao_v[pl.ds(base, _LANES)]
                cgc = cg_v[pl.ds(base, _LANES)]

                @plsc.parallel_loop(0, _C, 1, unroll=8)
                def c_body(c):
                    g1 = plsc.load_gather(x1_v, [i1 + c])
                    g2 = plsc.load_gather(x2_v, [i2 + c])
                    gw = plsc.load_gather(w_v, [iw + c])
                    t = g1 * g2 * gw * cgc
                    plsc.addupdate_scatter(out_v, [io + c], t)
                return c3
            lax.fori_loop(0, _NCHUNK, chunk_body, 0)

            pltpu.sync_copy(out_v, out_hbm.at[b])
            return carry
        lax.fori_loop(0, _BPW, batch_body, 0)

    return k(x1f, x2f, wf, cg, a1, a2, aw, ao)


def kernel(x1, x2, weight, CG_vals, l_ind_M1M2, M1, M2, M_ptr_M1M2):
    n_idx = jnp.arange(_NNZ, dtype=jnp.int32)
    seg = jnp.sum(n_idx[None, :] >= M_ptr_M1M2[1:_M, None],
                  axis=0, dtype=jnp.int32)
    a1 = M1 * _C
    a2 = M2 * _C
    aw = l_ind_M1M2 * _C
    ao = seg * _C

    x1f = x1.reshape(_B, _M * _C)
    x2f = x2.reshape(_B, _M * _C)
    wf = weight.reshape(_B, _NT * _C)

    out = _sc_tensor_product(x1f, x2f, wf, CG_vals, a1, a2, aw, ao)
    return out.reshape(_B, _M, _C)


# speedup vs baseline: 5.6878x; 4.5489x over previous
import functools

import jax
import jax.numpy as jnp
from jax import lax
from jax.experimental import pallas as pl
from jax.experimental.pallas import tpu as pltpu
from jax.experimental.pallas import tpu_sc as plsc

_B = 1024
_M = 16
_C = 128
_NNZ = 512
_NT = 34

_LANES = 16
_NW = 32
_BPW = _B // _NW
_NCHUNK = _NNZ // _LANES


def _sc_tensor_product(x1f, x2f, wf, cg, a1, a2, aw, ao):
    mesh = plsc.VectorSubcoreMesh(core_axis_name="c", subcore_axis_name="s")

    @functools.partial(
        pl.kernel,
        mesh=mesh,
        out_type=jax.ShapeDtypeStruct((_B, _M * _C), jnp.float32),
        compiler_params=pltpu.CompilerParams(needs_layout_passes=False),
        scratch_types=[
            pltpu.VMEM((_NNZ,), jnp.int32),
            pltpu.VMEM((_NNZ,), jnp.int32),
            pltpu.VMEM((_NNZ,), jnp.int32),
            pltpu.VMEM((_NNZ,), jnp.int32),
            pltpu.VMEM((_NNZ,), jnp.float32),
            pltpu.VMEM((_M * _C,), jnp.float32),
            pltpu.VMEM((_M * _C,), jnp.float32),
            pltpu.VMEM((_NT * _C,), jnp.float32),
            pltpu.VMEM((_M * _C,), jnp.float32),
        ],
    )
    def k(x1_hbm, x2_hbm, w_hbm, cg_hbm, a1_hbm, a2_hbm, aw_hbm, ao_hbm,
          out_hbm, a1_v, a2_v, aw_v, ao_v, cg_v, x1_v, x2_v, w_v, out_v):
        wid = lax.axis_index("c") * 16 + lax.axis_index("s")

        pltpu.sync_copy(a1_hbm, a1_v)
        pltpu.sync_copy(a2_hbm, a2_v)
        pltpu.sync_copy(aw_hbm, aw_v)
        pltpu.sync_copy(ao_hbm, ao_v)
        pltpu.sync_copy(cg_hbm, cg_v)

        def batch_body(i, carry):
            b = wid * _BPW + i
            pltpu.sync_copy(x1_hbm.at[b], x1_v)
            pltpu.sync_copy(x2_hbm.at[b], x2_v)
            pltpu.sync_copy(w_hbm.at[b], w_v)

            @plsc.parallel_loop(0, _M * _C, _LANES, unroll=4)
            def zero_body(kk):
                out_v[pl.ds(kk, _LANES)] = jnp.zeros((_LANES,), jnp.float32)

            @plsc.parallel_loop(0, _NNZ, _LANES)
            def n_body(base):
                i1v = a1_v[pl.ds(base, _LANES)]
                i2v = a2_v[pl.ds(base, _LANES)]
                iwv = aw_v[pl.ds(base, _LANES)]
                iov = ao_v[pl.ds(base, _LANES)]
                cgv = cg_v[pl.ds(base, _LANES)]
                for j in range(_LANES):
                    o1 = i1v[j]
                    o2 = i2v[j]
                    ow = iwv[j]
                    oo = iov[j]
                    cgs = cgv[j]
                    for kk in range(_C // _LANES):
                        g1 = x1_v[pl.ds(o1 + kk * _LANES, _LANES)]
                        g2 = x2_v[pl.ds(o2 + kk * _LANES, _LANES)]
                        gw = w_v[pl.ds(ow + kk * _LANES, _LANES)]
                        t = g1 * g2 * gw * cgs
                        plsc.addupdate(
                            out_v.at[pl.ds(oo + kk * _LANES, _LANES)], t)

            pltpu.sync_copy(out_v, out_hbm.at[b])
            return carry
        lax.fori_loop(0, _BPW, batch_body, 0)

    return k(x1f, x2f, wf, cg, a1, a2, aw, ao)


def kernel(x1, x2, weight, CG_vals, l_ind_M1M2, M1, M2, M_ptr_M1M2):
    n_idx = jnp.arange(_NNZ, dtype=jnp.int32)
    seg = jnp.sum(n_idx[None, :] >= M_ptr_M1M2[1:_M, None],
                  axis=0, dtype=jnp.int32)
    a1 = M1 * _C
    a2 = M2 * _C
    aw = l_ind_M1M2 * _C
    ao = seg * _C

    x1f = x1.reshape(_B, _M * _C)
    x2f = x2.reshape(_B, _M * _C)
    wf = weight.reshape(_B, _NT * _C)

    out = _sc_tensor_product(x1f, x2f, wf, CG_vals, a1, a2, aw, ao)
    return out.reshape(_B, _M, _C)


# speedup vs baseline: 14.3626x; 2.5252x over previous
import functools

import jax
import jax.numpy as jnp
from jax import lax
from jax.experimental import pallas as pl
from jax.experimental.pallas import tpu as pltpu
from jax.experimental.pallas import tpu_sc as plsc

_B = 1024
_M = 16
_C = 128
_NNZ = 512
_NT = 34

_LANES = 16
_NW = 32
_BPW = _B // _NW
_NCHUNK = _NNZ // _LANES


def _sc_tensor_product(x1f, x2f, wf, cg, p12, pwo):
    mesh = plsc.VectorSubcoreMesh(core_axis_name="c", subcore_axis_name="s")

    @functools.partial(
        pl.kernel,
        mesh=mesh,
        out_type=jax.ShapeDtypeStruct((_B, _M * _C), jnp.float32),
        compiler_params=pltpu.CompilerParams(needs_layout_passes=False),
        scratch_types=[
            pltpu.SMEM((_NNZ,), jnp.int32),
            pltpu.SMEM((_NNZ,), jnp.int32),
            pltpu.SMEM((_NNZ,), jnp.float32),
            pltpu.VMEM((_NNZ,), jnp.int32),
            pltpu.VMEM((_NNZ,), jnp.int32),
            pltpu.VMEM((_NNZ,), jnp.float32),
            pltpu.VMEM((_M * _C,), jnp.float32),
            pltpu.VMEM((_M * _C,), jnp.float32),
            pltpu.VMEM((_NT * _C,), jnp.float32),
            pltpu.VMEM((_M * _C,), jnp.float32),
        ],
    )
    def k(x1_hbm, x2_hbm, w_hbm, cg_hbm, p12_hbm, pwo_hbm,
          out_hbm, p12_s, pwo_s, cg_s, p12_b, pwo_b, cg_b,
          x1_v, x2_v, w_v, out_v):
        wid = lax.axis_index("c") * 16 + lax.axis_index("s")

        pltpu.sync_copy(p12_hbm, p12_b)
        pltpu.sync_copy(pwo_hbm, pwo_b)
        pltpu.sync_copy(cg_hbm, cg_b)

        @plsc.parallel_loop(0, _NNZ, _LANES)
        def fill_body(base):
            v12 = p12_b[pl.ds(base, _LANES)]
            vwo = pwo_b[pl.ds(base, _LANES)]
            vcg = cg_b[pl.ds(base, _LANES)]
            for j in range(_LANES):
                p12_s[base + j] = v12[j]
                pwo_s[base + j] = vwo[j]
                cg_s[base + j] = vcg[j]

        def batch_body(i, carry):
            b = wid * _BPW + i
            pltpu.sync_copy(x1_hbm.at[b], x1_v)
            pltpu.sync_copy(x2_hbm.at[b], x2_v)
            pltpu.sync_copy(w_hbm.at[b], w_v)

            @plsc.parallel_loop(0, _M * _C, _LANES, unroll=4)
            def zero_body(kk):
                out_v[pl.ds(kk, _LANES)] = jnp.zeros((_LANES,), jnp.float32)

            @plsc.parallel_loop(0, _NNZ, 1, unroll=2)
            def n_body(n):
                s12 = p12_s[n]
                swo = pwo_s[n]
                cgs = cg_s[n]
                o1 = s12 & 2047
                o2 = lax.shift_right_logical(s12, 11)
                ow = swo & 8191
                oo = lax.shift_right_logical(swo, 13)
                for kk in range(_C // _LANES):
                    g1 = x1_v[pl.ds(o1 + kk * _LANES, _LANES)]
                    g2 = x2_v[pl.ds(o2 + kk * _LANES, _LANES)]
                    gw = w_v[pl.ds(ow + kk * _LANES, _LANES)]
                    t = g1 * g2 * gw * cgs
                    plsc.addupdate(
                        out_v.at[pl.ds(oo + kk * _LANES, _LANES)], t)

            pltpu.sync_copy(out_v, out_hbm.at[b])
            return carry
        lax.fori_loop(0, _BPW, batch_body, 0)

    return k(x1f, x2f, wf, cg, p12, pwo)


def kernel(x1, x2, weight, CG_vals, l_ind_M1M2, M1, M2, M_ptr_M1M2):
    n_idx = jnp.arange(_NNZ, dtype=jnp.int32)
    seg = jnp.sum(n_idx[None, :] >= M_ptr_M1M2[1:_M, None],
                  axis=0, dtype=jnp.int32)
    a1 = M1 * _C
    a2 = M2 * _C
    aw = l_ind_M1M2 * _C
    ao = seg * _C
    p12 = a1 | (a2 << 11)
    pwo = aw | (ao << 13)

    x1f = x1.reshape(_B, _M * _C)
    x2f = x2.reshape(_B, _M * _C)
    wf = weight.reshape(_B, _NT * _C)

    out = _sc_tensor_product(x1f, x2f, wf, CG_vals, p12, pwo)
    return out.reshape(_B, _M, _C)
